# baseline (speedup 1.0000x reference)
import functools

import jax
import jax.numpy as jnp
from jax import lax
from jax.experimental import pallas as pl
from jax.experimental.pallas import tpu as pltpu
from jax.experimental.pallas import tpu_sc as plsc

N_NODES = 10000
N_EDGES = 320000
HID = 128
EED = 32
NUM_EDGE_LABELS = 256
NUM_NODE_LABELS = 4096
NUM_LAYERS = 3
TWO_48_MINUS_1 = float(2 ** 48 - 1)

NC = 2
NS = 16
NW = NC * NS
LANES = 16

E_PER_W = N_EDGES // NW
CHUNK = 40
N_CHUNKS = E_PER_W // CHUNK

COL_PERM = [32 * k + i for k in range(HID // 32)
            for i in list(range(0, 32, 2)) + list(range(1, 32, 2))]
STRIPE = 632
STRIPE_LAST = N_NODES - (NS - 1) * STRIPE

_sc_mesh = plsc.VectorSubcoreMesh(
    core_axis_name="c", subcore_axis_name="s", num_cores=NC, num_subcores=NS)


def _emb_gather_body(raw_hbm, emb_hbm, out_hbm, raw_v, idx_v, rows_v, sem):
    c = lax.axis_index("c")
    s = lax.axis_index("s")
    w = s * NC + c
    n_chunks = N_NODES // LANES
    n_iter = (n_chunks + NW - 1) // NW

    def body(i, carry):
        chunk = w + NW * i

        @pl.when(chunk < n_chunks)
        def _():
            base = pl.multiple_of(chunk * LANES, LANES)
            pltpu.sync_copy(raw_hbm.at[pl.ds(base, LANES)], raw_v)
            ids = lax.rem(raw_v[...].astype(jnp.int32),
                          jnp.int32(NUM_NODE_LABELS))
            idx_v[...] = ids
            pltpu.async_copy(emb_hbm.at[idx_v], rows_v, sem).wait()
            pltpu.sync_copy(rows_v, out_hbm.at[pl.ds(base, LANES)])
        return carry

    lax.fori_loop(0, n_iter, body, 0)


@functools.partial(
    pl.kernel,
    out_type=jax.ShapeDtypeStruct((N_NODES, HID), jnp.float32),
    mesh=_sc_mesh,
    scratch_types=[
        pltpu.VMEM((LANES,), jnp.float32),
        pltpu.VMEM((LANES,), jnp.int32),
        pltpu.VMEM((LANES, HID), jnp.float32),
        pltpu.SemaphoreType.DMA,
    ],
)
def _emb_gather(raw_hbm, emb_hbm, out_hbm, raw_v, idx_v, rows_v, sem):
    _emb_gather_body(raw_hbm, emb_hbm, out_hbm, raw_v, idx_v, rows_v, sem)


def _agg_body(x_hbm, t_hbm, src_hbm, dst_hbm, lab_hbm, zeros_hbm, out_hbm,
              sidx0, sidx1, sidx2, sidx3, lidx0, lidx1, lidx2, lidx3,
              didx0, didx1, didx2, didx3, xrows0, xrows1, trows0, trows1,
              mrows0, mrows1, agg_sh, t_sh,
              gsem0, gsem1, tsem0, tsem1, ssem0, ssem1,
              islsem0, islsem1, islsem2, islsem3,
              idsem0, idsem1, idsem2, idsem3):
    c = lax.axis_index("c")
    s = lax.axis_index("s")
    w = s * NC + c

    @pl.when(s == 0)
    def _():
        pltpu.sync_copy(t_hbm, t_sh)

    row0 = pl.multiple_of(s * STRIPE, 8)

    @pl.when(s < NS - 1)
    def _():
        pltpu.sync_copy(zeros_hbm.at[pl.ds(row0, STRIPE)],
                        agg_sh.at[pl.ds(row0, STRIPE)])

    @pl.when(s == NS - 1)
    def _():
        pltpu.sync_copy(zeros_hbm.at[pl.ds(row0, STRIPE_LAST)],
                        agg_sh.at[pl.ds(row0, STRIPE_LAST)])

    plsc.subcore_barrier()

    sidx = (sidx0, sidx1, sidx2, sidx3)
    lidx = (lidx0, lidx1, lidx2, lidx3)
    didx = (didx0, didx1, didx2, didx3)
    islsem = (islsem0, islsem1, islsem2, islsem3)
    idsem = (idsem0, idsem1, idsem2, idsem3)
    rbufs = ((xrows0, trows0, mrows0, gsem0, tsem0, ssem0),
             (xrows1, trows1, mrows1, gsem1, tsem1, ssem1))

    def issue_idx_sl(j, b4):
        pltpu.async_copy(src_hbm.at[w, j], sidx[b4], islsem[b4])
        pltpu.async_copy(lab_hbm.at[w, j], lidx[b4], islsem[b4])

    def wait_idx_sl(b4):
        pltpu.make_async_copy(src_hbm.at[0, 0], sidx[b4], islsem[b4]).wait()
        pltpu.make_async_copy(lab_hbm.at[0, 0], lidx[b4], islsem[b4]).wait()

    def issue_idx_d(j, b4):
        pltpu.async_copy(dst_hbm.at[w, j], didx[b4], idsem[b4])

    def wait_idx_d(b4):
        pltpu.make_async_copy(dst_hbm.at[0, 0], didx[b4], idsem[b4]).wait()

    def issue_gather(b4, p2):
        xr, tr, _, gs, ts, _ = rbufs[p2]
        pltpu.async_copy(x_hbm.at[sidx[b4].at[0]], xr, gs)
        pltpu.async_copy(t_sh.at[lidx[b4].at[0]], tr, ts)

    def wait_gather(p2):
        xr, tr, _, gs, ts, _ = rbufs[p2]
        pltpu.make_async_copy(x_hbm.at[sidx[0].at[0]], xr, gs).wait()
        pltpu.make_async_copy(t_sh.at[lidx[0].at[0]], tr, ts).wait()

    def compute(p2):
        xr, tr, mr, _, _, _ = rbufs[p2]

        def row_body(r, rc):
            for k in range(HID // (2 * LANES)):
                wx = xr[r, pl.ds(k * LANES, LANES)]
                lo = lax.bitcast_convert_type(wx << jnp.int32(16),
                                              jnp.float32)
                hi = lax.bitcast_convert_type(wx & jnp.int32(-65536),
                                              jnp.float32)
                cE = pl.ds(k * 2 * LANES, LANES)
                cO = pl.ds(k * 2 * LANES + LANES, LANES)
                mr[r, cE] = jnp.maximum(lo + tr[r, cE], 0.0)
                mr[r, cO] = jnp.maximum(hi + tr[r, cO], 0.0)
            return rc

        lax.fori_loop(0, CHUNK, row_body, 0)

    def issue_scatter(b4, p2):
        mr, ss = rbufs[p2][2], rbufs[p2][5]
        pltpu.async_copy(mr, agg_sh.at[didx[b4].at[0]], ss, add=True)

    def wait_scatter(p2):
        mr, ss = rbufs[p2][2], rbufs[p2][5]
        pltpu.make_async_copy(mr, agg_sh.at[didx[0].at[0]], ss).wait()

    pltpu.sync_copy(src_hbm.at[w, 0], sidx[0])
    pltpu.sync_copy(lab_hbm.at[w, 0], lidx[0])
    pltpu.sync_copy(src_hbm.at[w, 1], sidx[1])
    pltpu.sync_copy(lab_hbm.at[w, 1], lidx[1])
    pltpu.sync_copy(dst_hbm.at[w, 0], didx[0])
    pltpu.sync_copy(dst_hbm.at[w, 1], didx[1])
    issue_gather(0, 0)
    issue_gather(1, 1)
    issue_idx_sl(2, 2)
    issue_idx_sl(3, 3)

    for jj in (0, 1):
        p2 = jj % 2
        wait_gather(p2)
        compute(p2)
        wait_idx_sl((jj + 2) % 4)
        issue_gather((jj + 2) % 4, p2)
        issue_scatter(jj % 4, p2)
        issue_idx_sl(jj + 4, jj % 4)
        issue_idx_d(jj + 2, (jj + 2) % 4)

    def loop_body(i, carry):
        for p in range(4):
            j = 4 * i + 2 + p
            p2 = p % 2
            b4 = (2 + p) % 4
            bn = p
            wait_gather(p2)
            wait_scatter(p2)
            wait_idx_d(b4)
            compute(p2)
            wait_idx_sl(bn)
            issue_gather(bn, p2)
            issue_scatter(b4, p2)

            @pl.when(j + 4 < N_CHUNKS)
            def _():
                issue_idx_sl(j + 4, b4)

            issue_idx_d(j + 2, bn)
        return carry

    lax.fori_loop(0, (N_CHUNKS - 6) // 4, loop_body, 0)

    for jj in (N_CHUNKS - 4, N_CHUNKS - 3):
        p2 = jj % 2
        wait_gather(p2)
        wait_scatter(p2)
        wait_idx_d(jj % 4)
        compute(p2)
        wait_idx_sl((jj + 2) % 4)
        issue_gather((jj + 2) % 4, p2)
        issue_scatter(jj % 4, p2)
        issue_idx_d(jj + 2, (jj + 2) % 4)

    for jj in (N_CHUNKS - 2, N_CHUNKS - 1):
        p2 = jj % 2
        wait_gather(p2)
        wait_scatter(p2)
        wait_idx_d(jj % 4)
        compute(p2)
        issue_scatter(jj % 4, p2)
    wait_scatter(0)
    wait_scatter(1)

    plsc.subcore_barrier()

    @pl.when(s < NS - 1)
    def _():
        pltpu.sync_copy(agg_sh.at[pl.ds(row0, STRIPE)],
                        out_hbm.at[c, pl.ds(row0, STRIPE)])

    @pl.when(s == NS - 1)
    def _():
        pltpu.sync_copy(agg_sh.at[pl.ds(row0, STRIPE_LAST)],
                        out_hbm.at[c, pl.ds(row0, STRIPE_LAST)])


@functools.partial(
    pl.kernel,
    out_type=jax.ShapeDtypeStruct((NC, N_NODES, HID), jnp.float32),
    mesh=_sc_mesh,
    scratch_types=[
        pltpu.VMEM((1, CHUNK), jnp.int32),
        pltpu.VMEM((1, CHUNK), jnp.int32),
        pltpu.VMEM((1, CHUNK), jnp.int32),
        pltpu.VMEM((1, CHUNK), jnp.int32),
        pltpu.VMEM((1, CHUNK), jnp.int32),
        pltpu.VMEM((1, CHUNK), jnp.int32),
        pltpu.VMEM((1, CHUNK), jnp.int32),
        pltpu.VMEM((1, CHUNK), jnp.int32),
        pltpu.VMEM((1, CHUNK), jnp.int32),
        pltpu.VMEM((1, CHUNK), jnp.int32),
        pltpu.VMEM((1, CHUNK), jnp.int32),
        pltpu.VMEM((1, CHUNK), jnp.int32),
        pltpu.VMEM((CHUNK, HID // 2), jnp.int32),
        pltpu.VMEM((CHUNK, HID // 2), jnp.int32),
        pltpu.VMEM((CHUNK, HID), jnp.float32),
        pltpu.VMEM((CHUNK, HID), jnp.float32),
        pltpu.VMEM((CHUNK, HID), jnp.float32),
        pltpu.VMEM((CHUNK, HID), jnp.float32),
        pltpu.VMEM_SHARED((N_NODES, HID), jnp.float32),
        pltpu.VMEM_SHARED((NUM_EDGE_LABELS, HID), jnp.float32),
        pltpu.SemaphoreType.DMA,
        pltpu.SemaphoreType.DMA,
        pltpu.SemaphoreType.DMA,
        pltpu.SemaphoreType.DMA,
        pltpu.SemaphoreType.DMA,
        pltpu.SemaphoreType.DMA,
        pltpu.SemaphoreType.DMA,
        pltpu.SemaphoreType.DMA,
        pltpu.SemaphoreType.DMA,
        pltpu.SemaphoreType.DMA,
        pltpu.SemaphoreType.DMA,
        pltpu.SemaphoreType.DMA,
        pltpu.SemaphoreType.DMA,
        pltpu.SemaphoreType.DMA,
    ],
    compiler_params=pltpu.CompilerParams(use_tc_tiling_on_sc=False),
)
def _agg(*args):
    _agg_body(*args)


BN = 1024


def _tables_body(eemb, We1, be1, We2, be2, lW, lb, out):
    e = jnp.dot(eemb[...], We1[...], preferred_element_type=jnp.float32)
    e = jax.nn.relu(e + be1[...])
    e = jnp.dot(e, We2[...], preferred_element_type=jnp.float32) + be2[...]
    for l in range(NUM_LAYERS):
        t = jnp.dot(e, lW[l], preferred_element_type=jnp.float32) + lb[l]
        out[l] = t


def _edge_tables(edge_emb, We1, be1, We2, be2, lin_e_W, lin_e_b):
    return pl.pallas_call(
        _tables_body,
        out_shape=jax.ShapeDtypeStruct((NUM_LAYERS, NUM_EDGE_LABELS, HID),
                                       jnp.float32),
    )(edge_emb, We1, be1, We2, be2, lin_e_W, lin_e_b)


def _prep_body(raw, emb, Win, bin_, Wref, bref, out, outb):
    v = jnp.clip(raw[...] / TWO_48_MINUS_1, 0.0, 1.0)
    a = jax.nn.relu(v * Win[...] + bin_[...])
    x = jnp.dot(a, Wref[...], preferred_element_type=jnp.float32) + bref[...]
    x = x + emb[...]
    out[...] = x
    outb[...] = x.astype(jnp.bfloat16)


def _prep(raw2d, emb_x, W_in, b_in, W_ref, b_ref):
    grid = (N_NODES + BN - 1) // BN
    return pl.pallas_call(
        _prep_body,
        grid=(grid,),
        in_specs=[
            pl.BlockSpec((BN, 1), lambda i: (i, 0)),
            pl.BlockSpec((BN, HID), lambda i: (i, 0)),
            pl.BlockSpec((1, HID), lambda i: (0, 0)),
            pl.BlockSpec((1, HID), lambda i: (0, 0)),
            pl.BlockSpec((HID, HID), lambda i: (0, 0)),
            pl.BlockSpec((1, HID), lambda i: (0, 0)),
        ],
        out_specs=(pl.BlockSpec((BN, HID), lambda i: (i, 0)),
                   pl.BlockSpec((BN, HID), lambda i: (i, 0))),
        out_shape=(jax.ShapeDtypeStruct((N_NODES, HID), jnp.float32),
                   jax.ShapeDtypeStruct((N_NODES, HID), jnp.bfloat16)),
    )(raw2d, emb_x, W_in, b_in, W_ref, b_ref)


def _update_body(x, a0, a1, W1, W1p, b1, W2, b2, out, outb):
    agg = a0[...] + a1[...]
    h = (jnp.dot(x[...], W1[...], preferred_element_type=jnp.float32)
         + jnp.dot(agg, W1p[...], preferred_element_type=jnp.float32)
         + b1[...])
    h = jax.nn.relu(h)
    h = jnp.dot(h, W2[...], preferred_element_type=jnp.float32) + b2[...]
    h = jax.nn.relu(h)
    out[...] = h
    outb[...] = h.astype(jnp.bfloat16)


def _update(x, a0, a1, W1, W1p, b1, W2, b2):
    grid = (N_NODES + BN - 1) // BN
    return pl.pallas_call(
        _update_body,
        grid=(grid,),
        in_specs=[
            pl.BlockSpec((BN, HID), lambda i: (i, 0)),
            pl.BlockSpec((BN, HID), lambda i: (i, 0)),
            pl.BlockSpec((BN, HID), lambda i: (i, 0)),
            pl.BlockSpec((HID, HID), lambda i: (0, 0)),
            pl.BlockSpec((HID, HID), lambda i: (0, 0)),
            pl.BlockSpec((1, HID), lambda i: (0, 0)),
            pl.BlockSpec((HID, HID), lambda i: (0, 0)),
            pl.BlockSpec((1, HID), lambda i: (0, 0)),
        ],
        out_specs=(pl.BlockSpec((BN, HID), lambda i: (i, 0)),
                   pl.BlockSpec((BN, HID), lambda i: (i, 0))),
        out_shape=(jax.ShapeDtypeStruct((N_NODES, HID), jnp.float32),
                   jax.ShapeDtypeStruct((N_NODES, HID), jnp.bfloat16)),
    )(x, a0, a1, W1, W1p, b1, W2, b2)


def kernel(node_features, edge_index, edge_attr, W_in, b_in, W_ref, b_ref,
           node_emb, edge_emb, We1, be1, We2, be2, lin_e_W, lin_e_b,
           nn1_W, nn1_b, nn2_W, nn2_b):
    raw_flat = node_features.reshape(-1)
    src = edge_index[0].reshape(NW, N_CHUNKS, 1, CHUNK)
    dst = edge_index[1].reshape(NW, N_CHUNKS, 1, CHUNK)
    lab = edge_attr.astype(jnp.int32).reshape(NW, N_CHUNKS, 1, CHUNK)
    zeros = jnp.zeros((N_NODES, HID), jnp.float32)

    emb_x = _emb_gather(raw_flat, node_emb)
    tables = _edge_tables(edge_emb, We1, be1.reshape(1, EED), We2,
                          be2.reshape(1, EED), lin_e_W,
                          lin_e_b.reshape(NUM_LAYERS, 1, HID))
    perm = jnp.array(COL_PERM, dtype=jnp.int32)
    tables_p = tables[:, :, perm]
    x, xb = _prep(node_features, emb_x, W_in, b_in.reshape(1, HID), W_ref,
                  b_ref.reshape(1, HID))
    for l in range(NUM_LAYERS):
        xb32 = lax.bitcast_convert_type(
            xb.reshape(N_NODES, HID // 2, 2), jnp.int32)
        agg2 = _agg(xb32, tables_p[l], src, dst, lab, zeros)
        x, xb = _update(x, agg2[0], agg2[1], nn1_W[l], nn1_W[l][perm],
                        nn1_b[l].reshape(1, HID), nn2_W[l],
                        nn2_b[l].reshape(1, HID))
    return x


# speedup vs baseline: 1.5167x; 1.5167x over previous
import functools

import jax
import jax.numpy as jnp
from jax import lax
from jax.experimental import pallas as pl
from jax.experimental.pallas import tpu as pltpu
from jax.experimental.pallas import tpu_sc as plsc

N_NODES = 10000
N_EDGES = 320000
HID = 128
EED = 32
NUM_EDGE_LABELS = 256
NUM_NODE_LABELS = 4096
NUM_LAYERS = 3
TWO_48_MINUS_1 = float(2 ** 48 - 1)

NC = 2
NS = 16
NW = NC * NS
LANES = 16

E_PER_W = N_EDGES // NW
CHUNK = 40
N_CHUNKS = E_PER_W // CHUNK

STRIPE = 632
STRIPE_LAST = N_NODES - (NS - 1) * STRIPE

_sc_mesh = plsc.VectorSubcoreMesh(
    core_axis_name="c", subcore_axis_name="s", num_cores=NC, num_subcores=NS)


def _emb_gather_body(raw_hbm, emb_hbm, out_hbm, raw_v, idx_v, rows_v, sem):
    c = lax.axis_index("c")
    s = lax.axis_index("s")
    w = s * NC + c
    n_chunks = N_NODES // LANES
    n_iter = (n_chunks + NW - 1) // NW

    def body(i, carry):
        chunk = w + NW * i

        @pl.when(chunk < n_chunks)
        def _():
            base = pl.multiple_of(chunk * LANES, LANES)
            pltpu.sync_copy(raw_hbm.at[pl.ds(base, LANES)], raw_v)
            ids = lax.rem(raw_v[...].astype(jnp.int32),
                          jnp.int32(NUM_NODE_LABELS))
            idx_v[...] = ids
            pltpu.async_copy(emb_hbm.at[idx_v], rows_v, sem).wait()
            pltpu.sync_copy(rows_v, out_hbm.at[pl.ds(base, LANES)])
        return carry

    lax.fori_loop(0, n_iter, body, 0)


@functools.partial(
    pl.kernel,
    out_type=jax.ShapeDtypeStruct((N_NODES, HID), jnp.float32),
    mesh=_sc_mesh,
    scratch_types=[
        pltpu.VMEM((LANES,), jnp.float32),
        pltpu.VMEM((LANES,), jnp.int32),
        pltpu.VMEM((LANES, HID), jnp.float32),
        pltpu.SemaphoreType.DMA,
    ],
)
def _emb_gather(raw_hbm, emb_hbm, out_hbm, raw_v, idx_v, rows_v, sem):
    _emb_gather_body(raw_hbm, emb_hbm, out_hbm, raw_v, idx_v, rows_v, sem)


def _agg_body(x_hbm, t_hbm, src_hbm, dst_hbm, lab_hbm, zeros_hbm, out_hbm,
              sidx0, sidx1, sidx2, sidx3, lidx0, lidx1, lidx2, lidx3,
              didx0, didx1, didx2, didx3, xrows0, xrows1, trows0, trows1,
              mrows0, mrows1, agg_sh, t_sh,
              gsem0, gsem1, tsem0, tsem1, ssem0, ssem1,
              islsem0, islsem1, islsem2, islsem3,
              idsem0, idsem1, idsem2, idsem3):
    c = lax.axis_index("c")
    s = lax.axis_index("s")
    w = s * NC + c

    @pl.when(s == 0)
    def _():
        pltpu.sync_copy(t_hbm, t_sh)

    row0 = pl.multiple_of(s * STRIPE, 8)

    @pl.when(s < NS - 1)
    def _():
        pltpu.sync_copy(zeros_hbm.at[pl.ds(row0, STRIPE)],
                        agg_sh.at[pl.ds(row0, STRIPE)])

    @pl.when(s == NS - 1)
    def _():
        pltpu.sync_copy(zeros_hbm.at[pl.ds(row0, STRIPE_LAST)],
                        agg_sh.at[pl.ds(row0, STRIPE_LAST)])

    plsc.subcore_barrier()

    sidx = (sidx0, sidx1, sidx2, sidx3)
    lidx = (lidx0, lidx1, lidx2, lidx3)
    didx = (didx0, didx1, didx2, didx3)
    islsem = (islsem0, islsem1, islsem2, islsem3)
    idsem = (idsem0, idsem1, idsem2, idsem3)
    rbufs = ((xrows0, trows0, mrows0, gsem0, tsem0, ssem0),
             (xrows1, trows1, mrows1, gsem1, tsem1, ssem1))

    def issue_idx_sl(j, b4):
        pltpu.async_copy(src_hbm.at[w, j], sidx[b4], islsem[b4])
        pltpu.async_copy(lab_hbm.at[w, j], lidx[b4], islsem[b4])

    def wait_idx_sl(b4):
        pltpu.make_async_copy(src_hbm.at[0, 0], sidx[b4], islsem[b4]).wait()
        pltpu.make_async_copy(lab_hbm.at[0, 0], lidx[b4], islsem[b4]).wait()

    def issue_idx_d(j, b4):
        pltpu.async_copy(dst_hbm.at[w, j], didx[b4], idsem[b4])

    def wait_idx_d(b4):
        pltpu.make_async_copy(dst_hbm.at[0, 0], didx[b4], idsem[b4]).wait()

    def issue_gather(b4, p2):
        xr, tr, _, gs, ts, _ = rbufs[p2]
        pltpu.async_copy(x_hbm.at[sidx[b4].at[0]], xr, gs)
        pltpu.async_copy(t_sh.at[lidx[b4].at[0]], tr, ts)

    def wait_gather(p2):
        xr, tr, _, gs, ts, _ = rbufs[p2]
        pltpu.make_async_copy(x_hbm.at[sidx[0].at[0]], xr, gs).wait()
        pltpu.make_async_copy(t_sh.at[lidx[0].at[0]], tr, ts).wait()

    def compute(p2):
        xr, tr, mr, _, _, _ = rbufs[p2]

        @plsc.parallel_loop(0, CHUNK, 1, unroll=4)
        def _(r):
            for k in range(HID // LANES):
                sl = pl.ds(k * LANES, LANES)
                mr[r, sl] = jnp.maximum(xr[r, sl] + tr[r, sl], 0.0)

    def issue_scatter(b4, p2):
        mr, ss = rbufs[p2][2], rbufs[p2][5]
        pltpu.async_copy(mr, agg_sh.at[didx[b4].at[0]], ss, add=True)

    def wait_scatter(p2):
        mr, ss = rbufs[p2][2], rbufs[p2][5]
        pltpu.make_async_copy(mr, agg_sh.at[didx[0].at[0]], ss).wait()

    pltpu.sync_copy(src_hbm.at[w, 0], sidx[0])
    pltpu.sync_copy(lab_hbm.at[w, 0], lidx[0])
    pltpu.sync_copy(src_hbm.at[w, 1], sidx[1])
    pltpu.sync_copy(lab_hbm.at[w, 1], lidx[1])
    pltpu.sync_copy(dst_hbm.at[w, 0], didx[0])
    pltpu.sync_copy(dst_hbm.at[w, 1], didx[1])
    issue_gather(0, 0)
    issue_gather(1, 1)
    issue_idx_sl(2, 2)
    issue_idx_sl(3, 3)

    for jj in (0, 1):
        p2 = jj % 2
        wait_gather(p2)
        compute(p2)
        wait_idx_sl((jj + 2) % 4)
        issue_gather((jj + 2) % 4, p2)
        issue_scatter(jj % 4, p2)
        issue_idx_sl(jj + 4, jj % 4)
        issue_idx_d(jj + 2, (jj + 2) % 4)

    def loop_body(i, carry):
        for p in range(4):
            j = 4 * i + 2 + p
            p2 = p % 2
            b4 = (2 + p) % 4
            bn = p
            wait_gather(p2)
            wait_scatter(p2)
            wait_idx_d(b4)
            compute(p2)
            wait_idx_sl(bn)
            issue_gather(bn, p2)
            issue_scatter(b4, p2)

            @pl.when(j + 4 < N_CHUNKS)
            def _():
                issue_idx_sl(j + 4, b4)

            issue_idx_d(j + 2, bn)
        return carry

    lax.fori_loop(0, (N_CHUNKS - 6) // 4, loop_body, 0)

    for jj in (N_CHUNKS - 4, N_CHUNKS - 3):
        p2 = jj % 2
        wait_gather(p2)
        wait_scatter(p2)
        wait_idx_d(jj % 4)
        compute(p2)
        wait_idx_sl((jj + 2) % 4)
        issue_gather((jj + 2) % 4, p2)
        issue_scatter(jj % 4, p2)
        issue_idx_d(jj + 2, (jj + 2) % 4)

    for jj in (N_CHUNKS - 2, N_CHUNKS - 1):
        p2 = jj % 2
        wait_gather(p2)
        wait_scatter(p2)
        wait_idx_d(jj % 4)
        compute(p2)
        issue_scatter(jj % 4, p2)
    wait_scatter(0)
    wait_scatter(1)

    plsc.subcore_barrier()

    @pl.when(s < NS - 1)
    def _():
        pltpu.sync_copy(agg_sh.at[pl.ds(row0, STRIPE)],
                        out_hbm.at[c, pl.ds(row0, STRIPE)])

    @pl.when(s == NS - 1)
    def _():
        pltpu.sync_copy(agg_sh.at[pl.ds(row0, STRIPE_LAST)],
                        out_hbm.at[c, pl.ds(row0, STRIPE_LAST)])


@functools.partial(
    pl.kernel,
    out_type=jax.ShapeDtypeStruct((NC, N_NODES, HID), jnp.float32),
    mesh=_sc_mesh,
    scratch_types=[
        pltpu.VMEM((1, CHUNK), jnp.int32),
        pltpu.VMEM((1, CHUNK), jnp.int32),
        pltpu.VMEM((1, CHUNK), jnp.int32),
        pltpu.VMEM((1, CHUNK), jnp.int32),
        pltpu.VMEM((1, CHUNK), jnp.int32),
        pltpu.VMEM((1, CHUNK), jnp.int32),
        pltpu.VMEM((1, CHUNK), jnp.int32),
        pltpu.VMEM((1, CHUNK), jnp.int32),
        pltpu.VMEM((1, CHUNK), jnp.int32),
        pltpu.VMEM((1, CHUNK), jnp.int32),
        pltpu.VMEM((1, CHUNK), jnp.int32),
        pltpu.VMEM((1, CHUNK), jnp.int32),
        pltpu.VMEM((CHUNK, HID), jnp.float32),
        pltpu.VMEM((CHUNK, HID), jnp.float32),
        pltpu.VMEM((CHUNK, HID), jnp.float32),
        pltpu.VMEM((CHUNK, HID), jnp.float32),
        pltpu.VMEM((CHUNK, HID), jnp.float32),
        pltpu.VMEM((CHUNK, HID), jnp.float32),
        pltpu.VMEM_SHARED((N_NODES, HID), jnp.float32),
        pltpu.VMEM_SHARED((NUM_EDGE_LABELS, HID), jnp.float32),
        pltpu.SemaphoreType.DMA,
        pltpu.SemaphoreType.DMA,
        pltpu.SemaphoreType.DMA,
        pltpu.SemaphoreType.DMA,
        pltpu.SemaphoreType.DMA,
        pltpu.SemaphoreType.DMA,
        pltpu.SemaphoreType.DMA,
        pltpu.SemaphoreType.DMA,
        pltpu.SemaphoreType.DMA,
        pltpu.SemaphoreType.DMA,
        pltpu.SemaphoreType.DMA,
        pltpu.SemaphoreType.DMA,
        pltpu.SemaphoreType.DMA,
        pltpu.SemaphoreType.DMA,
    ],
)
def _agg(*args):
    _agg_body(*args)


BN = 1024


def _tables_body(eemb, We1, be1, We2, be2, lW, lb, out):
    e = jnp.dot(eemb[...], We1[...], preferred_element_type=jnp.float32)
    e = jax.nn.relu(e + be1[...])
    e = jnp.dot(e, We2[...], preferred_element_type=jnp.float32) + be2[...]
    for l in range(NUM_LAYERS):
        t = jnp.dot(e, lW[l], preferred_element_type=jnp.float32) + lb[l]
        out[l] = t


def _edge_tables(edge_emb, We1, be1, We2, be2, lin_e_W, lin_e_b):
    return pl.pallas_call(
        _tables_body,
        out_shape=jax.ShapeDtypeStruct((NUM_LAYERS, NUM_EDGE_LABELS, HID),
                                       jnp.float32),
    )(edge_emb, We1, be1, We2, be2, lin_e_W, lin_e_b)


def _prep_body(raw, emb, Win, bin_, Wref, bref, out):
    v = jnp.clip(raw[...] / TWO_48_MINUS_1, 0.0, 1.0)
    a = jax.nn.relu(v * Win[...] + bin_[...])
    x = jnp.dot(a, Wref[...], preferred_element_type=jnp.float32) + bref[...]
    out[...] = x + emb[...]


def _prep(raw2d, emb_x, W_in, b_in, W_ref, b_ref):
    grid = (N_NODES + BN - 1) // BN
    return pl.pallas_call(
        _prep_body,
        grid=(grid,),
        in_specs=[
            pl.BlockSpec((BN, 1), lambda i: (i, 0)),
            pl.BlockSpec((BN, HID), lambda i: (i, 0)),
            pl.BlockSpec((1, HID), lambda i: (0, 0)),
            pl.BlockSpec((1, HID), lambda i: (0, 0)),
            pl.BlockSpec((HID, HID), lambda i: (0, 0)),
            pl.BlockSpec((1, HID), lambda i: (0, 0)),
        ],
        out_specs=pl.BlockSpec((BN, HID), lambda i: (i, 0)),
        out_shape=jax.ShapeDtypeStruct((N_NODES, HID), jnp.float32),
    )(raw2d, emb_x, W_in, b_in, W_ref, b_ref)


def _update_body(x, a0, a1, W1, b1, W2, b2, out):
    xa = x[...] + a0[...] + a1[...]
    h = jnp.dot(xa, W1[...], preferred_element_type=jnp.float32) + b1[...]
    h = jax.nn.relu(h)
    h = jnp.dot(h, W2[...], preferred_element_type=jnp.float32) + b2[...]
    out[...] = jax.nn.relu(h)


def _update(x, a0, a1, W1, b1, W2, b2):
    grid = (N_NODES + BN - 1) // BN
    return pl.pallas_call(
        _update_body,
        grid=(grid,),
        in_specs=[
            pl.BlockSpec((BN, HID), lambda i: (i, 0)),
            pl.BlockSpec((BN, HID), lambda i: (i, 0)),
            pl.BlockSpec((BN, HID), lambda i: (i, 0)),
            pl.BlockSpec((HID, HID), lambda i: (0, 0)),
            pl.BlockSpec((1, HID), lambda i: (0, 0)),
            pl.BlockSpec((HID, HID), lambda i: (0, 0)),
            pl.BlockSpec((1, HID), lambda i: (0, 0)),
        ],
        out_specs=pl.BlockSpec((BN, HID), lambda i: (i, 0)),
        out_shape=jax.ShapeDtypeStruct((N_NODES, HID), jnp.float32),
    )(x, a0, a1, W1, b1, W2, b2)


def kernel(node_features, edge_index, edge_attr, W_in, b_in, W_ref, b_ref,
           node_emb, edge_emb, We1, be1, We2, be2, lin_e_W, lin_e_b,
           nn1_W, nn1_b, nn2_W, nn2_b):
    raw_flat = node_features.reshape(-1)
    src = edge_index[0].reshape(NW, N_CHUNKS, 1, CHUNK)
    dst = edge_index[1].reshape(NW, N_CHUNKS, 1, CHUNK)
    lab = edge_attr.astype(jnp.int32).reshape(NW, N_CHUNKS, 1, CHUNK)
    zeros = jnp.zeros((N_NODES, HID), jnp.float32)

    emb_x = _emb_gather(raw_flat, node_emb)
    tables = _edge_tables(edge_emb, We1, be1.reshape(1, EED), We2,
                          be2.reshape(1, EED), lin_e_W,
                          lin_e_b.reshape(NUM_LAYERS, 1, HID))
    x = _prep(node_features, emb_x, W_in, b_in.reshape(1, HID), W_ref,
              b_ref.reshape(1, HID))
    for l in range(NUM_LAYERS):
        agg2 = _agg(x, tables[l], src, dst, lab, zeros)
        x = _update(x, agg2[0], agg2[1], nn1_W[l],
                    nn1_b[l].reshape(1, HID), nn2_W[l],
                    nn2_b[l].reshape(1, HID))
    return x


# speedup vs baseline: 1.5439x; 1.0179x over previous
import functools

import jax
import jax.numpy as jnp
from jax import lax
from jax.experimental import pallas as pl
from jax.experimental.pallas import tpu as pltpu
from jax.experimental.pallas import tpu_sc as plsc

N_NODES = 10000
N_EDGES = 320000
HID = 128
EED = 32
NUM_EDGE_LABELS = 256
NUM_NODE_LABELS = 4096
NUM_LAYERS = 3
TWO_48_MINUS_1 = float(2 ** 48 - 1)

NC = 2
NS = 16
NW = NC * NS
LANES = 16

E_PER_W = N_EDGES // NW
CHUNK = 40
N_CHUNKS = E_PER_W // CHUNK

STRIPE = 632
STRIPE_LAST = N_NODES - (NS - 1) * STRIPE

_sc_mesh = plsc.VectorSubcoreMesh(
    core_axis_name="c", subcore_axis_name="s", num_cores=NC, num_subcores=NS)


def _emb_gather_body(raw_hbm, emb_hbm, out_hbm, raw_v, idx_v, rows_v, sem):
    c = lax.axis_index("c")
    s = lax.axis_index("s")
    w = s * NC + c
    n_chunks = N_NODES // LANES
    n_iter = (n_chunks + NW - 1) // NW

    def body(i, carry):
        chunk = w + NW * i

        @pl.when(chunk < n_chunks)
        def _():
            base = pl.multiple_of(chunk * LANES, LANES)
            pltpu.sync_copy(raw_hbm.at[pl.ds(base, LANES)], raw_v)
            ids = lax.rem(raw_v[...].astype(jnp.int32),
                          jnp.int32(NUM_NODE_LABELS))
            idx_v[...] = ids
            pltpu.async_copy(emb_hbm.at[idx_v], rows_v, sem).wait()
            pltpu.sync_copy(rows_v, out_hbm.at[pl.ds(base, LANES)])
        return carry

    lax.fori_loop(0, n_iter, body, 0)


@functools.partial(
    pl.kernel,
    out_type=jax.ShapeDtypeStruct((N_NODES, HID), jnp.float32),
    mesh=_sc_mesh,
    scratch_types=[
        pltpu.VMEM((LANES,), jnp.float32),
        pltpu.VMEM((LANES,), jnp.int32),
        pltpu.VMEM((LANES, HID), jnp.float32),
        pltpu.SemaphoreType.DMA,
    ],
)
def _emb_gather(raw_hbm, emb_hbm, out_hbm, raw_v, idx_v, rows_v, sem):
    _emb_gather_body(raw_hbm, emb_hbm, out_hbm, raw_v, idx_v, rows_v, sem)


def _agg_body(x_hbm, t_hbm, src_hbm, dst_hbm, lab_hbm, zeros_hbm, out_hbm,
              sidx0, sidx1, sidx2, sidx3, lidx0, lidx1, lidx2, lidx3,
              didx0, didx1, didx2, didx3, xrows0, xrows1, trows0, trows1,
              mrows0, mrows1, agg_sh, t_sh,
              gsem0, gsem1, tsem0, tsem1, ssem0, ssem1,
              islsem0, islsem1, islsem2, islsem3,
              idsem0, idsem1, idsem2, idsem3):
    c = lax.axis_index("c")
    s = lax.axis_index("s")
    w = s * NC + c

    @pl.when(s == 0)
    def _():
        pltpu.sync_copy(t_hbm, t_sh)

    row0 = pl.multiple_of(s * STRIPE, 8)

    @pl.when(s < NS - 1)
    def _():
        pltpu.sync_copy(zeros_hbm.at[pl.ds(row0, STRIPE)],
                        agg_sh.at[pl.ds(row0, STRIPE)])

    @pl.when(s == NS - 1)
    def _():
        pltpu.sync_copy(zeros_hbm.at[pl.ds(row0, STRIPE_LAST)],
                        agg_sh.at[pl.ds(row0, STRIPE_LAST)])

    plsc.subcore_barrier()

    sidx = (sidx0, sidx1, sidx2, sidx3)
    lidx = (lidx0, lidx1, lidx2, lidx3)
    didx = (didx0, didx1, didx2, didx3)
    islsem = (islsem0, islsem1, islsem2, islsem3)
    idsem = (idsem0, idsem1, idsem2, idsem3)
    rbufs = ((xrows0, trows0, mrows0, gsem0, tsem0, ssem0),
             (xrows1, trows1, mrows1, gsem1, tsem1, ssem1))

    def issue_idx_sl(j, b4):
        pltpu.async_copy(src_hbm.at[w, j], sidx[b4], islsem[b4])
        pltpu.async_copy(lab_hbm.at[w, j], lidx[b4], islsem[b4])

    def wait_idx_sl(b4):
        pltpu.make_async_copy(src_hbm.at[0, 0], sidx[b4], islsem[b4]).wait()
        pltpu.make_async_copy(lab_hbm.at[0, 0], lidx[b4], islsem[b4]).wait()

    def issue_idx_d(j, b4):
        pltpu.async_copy(dst_hbm.at[w, j], didx[b4], idsem[b4])

    def wait_idx_d(b4):
        pltpu.make_async_copy(dst_hbm.at[0, 0], didx[b4], idsem[b4]).wait()

    def issue_gather(b4, p2):
        xr, tr, _, gs, ts, _ = rbufs[p2]
        pltpu.async_copy(x_hbm.at[sidx[b4].at[0]], xr, gs)
        pltpu.async_copy(t_sh.at[lidx[b4].at[0]], tr, ts)

    def wait_gather(p2):
        xr, tr, _, gs, ts, _ = rbufs[p2]
        pltpu.make_async_copy(x_hbm.at[sidx[0].at[0]], xr, gs).wait()
        pltpu.make_async_copy(t_sh.at[lidx[0].at[0]], tr, ts).wait()

    def compute(p2):
        xr, tr, mr, _, _, _ = rbufs[p2]

        def row_body(r, rc):
            for k in range(HID // LANES):
                sl = pl.ds(k * LANES, LANES)
                mr[r, sl] = jnp.maximum(xr[r, sl] + tr[r, sl], 0.0)
            return rc

        lax.fori_loop(0, CHUNK, row_body, 0)

    def issue_scatter(b4, p2):
        mr, ss = rbufs[p2][2], rbufs[p2][5]
        pltpu.async_copy(mr, agg_sh.at[didx[b4].at[0]], ss, add=True)

    def wait_scatter(p2):
        mr, ss = rbufs[p2][2], rbufs[p2][5]
        pltpu.make_async_copy(mr, agg_sh.at[didx[0].at[0]], ss).wait()

    pltpu.sync_copy(src_hbm.at[w, 0], sidx[0])
    pltpu.sync_copy(lab_hbm.at[w, 0], lidx[0])
    pltpu.sync_copy(src_hbm.at[w, 1], sidx[1])
    pltpu.sync_copy(lab_hbm.at[w, 1], lidx[1])
    pltpu.sync_copy(dst_hbm.at[w, 0], didx[0])
    pltpu.sync_copy(dst_hbm.at[w, 1], didx[1])
    issue_gather(0, 0)
    issue_gather(1, 1)
    issue_idx_sl(2, 2)
    issue_idx_sl(3, 3)

    for jj in (0, 1):
        p2 = jj % 2
        wait_gather(p2)
        compute(p2)
        wait_idx_sl((jj + 2) % 4)
        issue_gather((jj + 2) % 4, p2)
        issue_scatter(jj % 4, p2)
        issue_idx_sl(jj + 4, jj % 4)
        issue_idx_d(jj + 2, (jj + 2) % 4)

    def loop_body(i, carry):
        for p in range(4):
            j = 4 * i + 2 + p
            p2 = p % 2
            b4 = (2 + p) % 4
            bn = p
            wait_gather(p2)
            wait_scatter(p2)
            wait_idx_d(b4)
            compute(p2)
            wait_idx_sl(bn)
            issue_gather(bn, p2)
            issue_scatter(b4, p2)

            @pl.when(j + 4 < N_CHUNKS)
            def _():
                issue_idx_sl(j + 4, b4)

            issue_idx_d(j + 2, bn)
        return carry

    lax.fori_loop(0, (N_CHUNKS - 6) // 4, loop_body, 0)

    for jj in (N_CHUNKS - 4, N_CHUNKS - 3):
        p2 = jj % 2
        wait_gather(p2)
        wait_scatter(p2)
        wait_idx_d(jj % 4)
        compute(p2)
        wait_idx_sl((jj + 2) % 4)
        issue_gather((jj + 2) % 4, p2)
        issue_scatter(jj % 4, p2)
        issue_idx_d(jj + 2, (jj + 2) % 4)

    for jj in (N_CHUNKS - 2, N_CHUNKS - 1):
        p2 = jj % 2
        wait_gather(p2)
        wait_scatter(p2)
        wait_idx_d(jj % 4)
        compute(p2)
        issue_scatter(jj % 4, p2)
    wait_scatter(0)
    wait_scatter(1)

    plsc.subcore_barrier()

    @pl.when(s < NS - 1)
    def _():
        pltpu.sync_copy(agg_sh.at[pl.ds(row0, STRIPE)],
                        out_hbm.at[c, pl.ds(row0, STRIPE)])

    @pl.when(s == NS - 1)
    def _():
        pltpu.sync_copy(agg_sh.at[pl.ds(row0, STRIPE_LAST)],
                        out_hbm.at[c, pl.ds(row0, STRIPE_LAST)])


@functools.partial(
    pl.kernel,
    out_type=jax.ShapeDtypeStruct((NC, N_NODES, HID), jnp.float32),
    mesh=_sc_mesh,
    scratch_types=[
        pltpu.VMEM((1, CHUNK), jnp.int32),
        pltpu.VMEM((1, CHUNK), jnp.int32),
        pltpu.VMEM((1, CHUNK), jnp.int32),
        pltpu.VMEM((1, CHUNK), jnp.int32),
        pltpu.VMEM((1, CHUNK), jnp.int32),
        pltpu.VMEM((1, CHUNK), jnp.int32),
        pltpu.VMEM((1, CHUNK), jnp.int32),
        pltpu.VMEM((1, CHUNK), jnp.int32),
        pltpu.VMEM((1, CHUNK), jnp.int32),
        pltpu.VMEM((1, CHUNK), jnp.int32),
        pltpu.VMEM((1, CHUNK), jnp.int32),
        pltpu.VMEM((1, CHUNK), jnp.int32),
        pltpu.VMEM((CHUNK, HID), jnp.float32),
        pltpu.VMEM((CHUNK, HID), jnp.float32),
        pltpu.VMEM((CHUNK, HID), jnp.float32),
        pltpu.VMEM((CHUNK, HID), jnp.float32),
        pltpu.VMEM((CHUNK, HID), jnp.float32),
        pltpu.VMEM((CHUNK, HID), jnp.float32),
        pltpu.VMEM_SHARED((N_NODES, HID), jnp.float32),
        pltpu.VMEM_SHARED((NUM_EDGE_LABELS, HID), jnp.float32),
        pltpu.SemaphoreType.DMA,
        pltpu.SemaphoreType.DMA,
        pltpu.SemaphoreType.DMA,
        pltpu.SemaphoreType.DMA,
        pltpu.SemaphoreType.DMA,
        pltpu.SemaphoreType.DMA,
        pltpu.SemaphoreType.DMA,
        pltpu.SemaphoreType.DMA,
        pltpu.SemaphoreType.DMA,
        pltpu.SemaphoreType.DMA,
        pltpu.SemaphoreType.DMA,
        pltpu.SemaphoreType.DMA,
        pltpu.SemaphoreType.DMA,
        pltpu.SemaphoreType.DMA,
    ],
)
def _agg(*args):
    _agg_body(*args)


BN = 1024


def _tables_body(eemb, We1, be1, We2, be2, lW, lb, out):
    e = jnp.dot(eemb[...], We1[...], preferred_element_type=jnp.float32)
    e = jax.nn.relu(e + be1[...])
    e = jnp.dot(e, We2[...], preferred_element_type=jnp.float32) + be2[...]
    for l in range(NUM_LAYERS):
        t = jnp.dot(e, lW[l], preferred_element_type=jnp.float32) + lb[l]
        out[l] = t


def _edge_tables(edge_emb, We1, be1, We2, be2, lin_e_W, lin_e_b):
    return pl.pallas_call(
        _tables_body,
        out_shape=jax.ShapeDtypeStruct((NUM_LAYERS, NUM_EDGE_LABELS, HID),
                                       jnp.float32),
    )(edge_emb, We1, be1, We2, be2, lin_e_W, lin_e_b)


def _prep_body(raw, emb, Win, bin_, Wref, bref, out):
    v = jnp.clip(raw[...] / TWO_48_MINUS_1, 0.0, 1.0)
    a = jax.nn.relu(v * Win[...] + bin_[...])
    x = jnp.dot(a, Wref[...], preferred_element_type=jnp.float32) + bref[...]
    out[...] = x + emb[...]


def _prep(raw2d, emb_x, W_in, b_in, W_ref, b_ref):
    grid = (N_NODES + BN - 1) // BN
    return pl.pallas_call(
        _prep_body,
        grid=(grid,),
        in_specs=[
            pl.BlockSpec((BN, 1), lambda i: (i, 0)),
            pl.BlockSpec((BN, HID), lambda i: (i, 0)),
            pl.BlockSpec((1, HID), lambda i: (0, 0)),
            pl.BlockSpec((1, HID), lambda i: (0, 0)),
            pl.BlockSpec((HID, HID), lambda i: (0, 0)),
            pl.BlockSpec((1, HID), lambda i: (0, 0)),
        ],
        out_specs=pl.BlockSpec((BN, HID), lambda i: (i, 0)),
        out_shape=jax.ShapeDtypeStruct((N_NODES, HID), jnp.float32),
    )(raw2d, emb_x, W_in, b_in, W_ref, b_ref)


def _update_body(x, a0, a1, W1, b1, W2, b2, out):
    xa = x[...] + a0[...] + a1[...]
    h = jnp.dot(xa, W1[...], preferred_element_type=jnp.float32) + b1[...]
    h = jax.nn.relu(h)
    h = jnp.dot(h, W2[...], preferred_element_type=jnp.float32) + b2[...]
    out[...] = jax.nn.relu(h)


def _update(x, a0, a1, W1, b1, W2, b2):
    grid = (N_NODES + BN - 1) // BN
    return pl.pallas_call(
        _update_body,
        grid=(grid,),
        in_specs=[
            pl.BlockSpec((BN, HID), lambda i: (i, 0)),
            pl.BlockSpec((BN, HID), lambda i: (i, 0)),
            pl.BlockSpec((BN, HID), lambda i: (i, 0)),
            pl.BlockSpec((HID, HID), lambda i: (0, 0)),
            pl.BlockSpec((1, HID), lambda i: (0, 0)),
            pl.BlockSpec((HID, HID), lambda i: (0, 0)),
            pl.BlockSpec((1, HID), lambda i: (0, 0)),
        ],
        out_specs=pl.BlockSpec((BN, HID), lambda i: (i, 0)),
        out_shape=jax.ShapeDtypeStruct((N_NODES, HID), jnp.float32),
    )(x, a0, a1, W1, b1, W2, b2)


def kernel(node_features, edge_index, edge_attr, W_in, b_in, W_ref, b_ref,
           node_emb, edge_emb, We1, be1, We2, be2, lin_e_W, lin_e_b,
           nn1_W, nn1_b, nn2_W, nn2_b):
    raw_flat = node_features.reshape(-1)
    src = edge_index[0].reshape(NW, N_CHUNKS, 1, CHUNK)
    dst = edge_index[1].reshape(NW, N_CHUNKS, 1, CHUNK)
    lab = edge_attr.astype(jnp.int32).reshape(NW, N_CHUNKS, 1, CHUNK)
    zeros = jnp.zeros((N_NODES, HID), jnp.float32)

    emb_x = _emb_gather(raw_flat, node_emb)
    tables = _edge_tables(edge_emb, We1, be1.reshape(1, EED), We2,
                          be2.reshape(1, EED), lin_e_W,
                          lin_e_b.reshape(NUM_LAYERS, 1, HID))
    x = _prep(node_features, emb_x, W_in, b_in.reshape(1, HID), W_ref,
              b_ref.reshape(1, HID))
    for l in range(NUM_LAYERS):
        agg2 = _agg(x, tables[l], src, dst, lab, zeros)
        x = _update(x, agg2[0], agg2[1], nn1_W[l],
                    nn1_b[l].reshape(1, HID), nn2_W[l],
                    nn2_b[l].reshape(1, HID))
    return x
